# baseline (speedup 1.0000x reference)
import functools

import jax
import jax.numpy as jnp
from jax import lax
from jax.experimental import pallas as pl
from jax.experimental.pallas import tpu as pltpu
from jax.experimental.pallas import tpu_sc as plsc

B, P, T = 16, 4000, 32
LINED = 72
BIG = 100000000.0
NCHUNK = P // 16


def _cost_body(scal_ref, logits_ref, reg_ref, lines_ref, tlines_ref, tscv_ref,
               masksv_ref, cost_ref, kk_ref, ious_sc):
    wm1 = scal_ref[0, 0]
    wf = scal_ref[0, 1]
    kconst = scal_ref[0, 2]

    lg = logits_ref[0]
    x0 = lg[0:1, :]
    x1 = lg[1:2, :]
    m = jnp.maximum(x0, x1)
    e0 = jnp.exp(x0 - m)
    e1 = jnp.exp(x1 - m)
    p1 = e1 / (e0 + e1)
    c3 = 3.0 * (-jnp.log(jnp.maximum(p1, 1e-8)))

    p2 = reg_ref[0, 0:1, :]
    p3 = reg_ref[0, 1:2, :]
    p4 = reg_ref[0, 2:3, :]
    p2b = jnp.broadcast_to(p2, (8, P))
    p3b = jnp.broadcast_to(p3, (8, P))
    p4b = jnp.broadcast_to(p4, (8, P))
    c3b = jnp.broadcast_to(c3, (8, P))
    lb = lines_ref[0]

    num_gt = jnp.sum(masksv_ref[0])

    def reg3(g8):
        t3c = tscv_ref[0, pl.ds(g8, 8), 0:1]
        t2c = tscv_ref[0, pl.ds(g8, 8), 1:2]
        t4c = tscv_ref[0, pl.ds(g8, 8), 2:3]
        cx = jnp.abs(p3b - t3c)
        cy = jnp.abs(p2b - t2c)
        ct = jnp.abs(p4b - t4c)
        return 3.0 * ((cx + cy) + ct)

    di8 = jax.lax.broadcasted_iota(jnp.int32, (LINED, 8), 0) // 9
    gi8 = jax.lax.broadcasted_iota(jnp.int32, (LINED, 8), 1)
    g0 = jnp.where(di8 == gi8, 1.0, 0.0)
    pabs = jax.lax.dot_general(
        jnp.ones((1, LINED), jnp.bfloat16), jnp.abs(lb),
        (((1,), (0,)), ((), ())),
        preferred_element_type=jnp.float32)

    def fast_body(gi, okf):
        g8 = gi * 8
        cols = []
        gs_rows = []
        a_rows = []
        for j in range(8):
            gl = tlines_ref[0, g8 + j]
            gls = gl * wm1
            inv = (gls < 0.0) | (gls >= wf)
            v = jnp.where(inv, 0.0, 1.0)
            cols.append(v * g0)
            vgl = v * gl
            gs_rows.append(jnp.sum(vgl.reshape(8, 9, 1), axis=1))
            a_rows.append((30.0 * jnp.sum(v) / wm1).reshape(1, 1))
        lhs = jnp.concatenate(cols, axis=1).astype(jnp.bfloat16)
        d64 = jax.lax.dot_general(
            lhs, lb, (((0,), (0,)), ((), ())),
            preferred_element_type=jnp.float32)
        gs64 = jnp.concatenate(gs_rows, axis=0)
        bnd = jnp.sum(jnp.abs(d64 - gs64).reshape(8, 8, P), axis=1)
        a8u = jnp.concatenate(a_rows, axis=0)
        rhs = 1.05 * a8u + (0.004 * pabs + 0.01)
        ok = jnp.min(jnp.where(bnd >= rhs, 1.0, 0.0))
        m8 = tscv_ref[0, pl.ds(g8, 8), 3:4]
        valid8 = m8 > 0.0
        i38 = 3.0 * jnp.where(valid8, kconst, 0.0)
        tot8 = ((c3b + reg3(g8)) + i38) + (100000.0 *
                                           jnp.where(valid8, 0.0, 1.0))
        cost_ref[0, pl.ds(g8, 8), :] = tot8
        return jnp.minimum(okf, ok)

    allok = jax.lax.fori_loop(0, T // 8, fast_body, jnp.float32(1.0),
                              unroll=False) > 0.5

    @pl.when(jnp.logical_not(allok))
    def _exact_fill():
        def g_body(gi, _):
            g8 = gi * 8
            s_rows = []
            nv_rows = []
            for j in range(8):
                gl = tlines_ref[0, g8 + j]
                gls = gl * wm1
                inv = (gls < 0.0) | (gls >= wf)
                v = jnp.where(inv, 0.0, 1.0)
                ad = jnp.abs(lb - gl.astype(jnp.bfloat16))
                sj = jax.lax.dot_general(
                    v.astype(jnp.bfloat16), ad,
                    (((0,), (0,)), ((), ())),
                    preferred_element_type=jnp.float32)
                s_rows.append(sj)
                nv_rows.append(jnp.sum(v).reshape(1, 1))
            s8 = wm1 * jnp.concatenate(s_rows, axis=0)
            nv8 = jnp.concatenate(nv_rows, axis=0)
            a8 = 30.0 * nv8
            li8 = (a8 - s8) / ((a8 + s8) + 1e-9)
            m8 = tscv_ref[0, pl.ds(g8, 8), 3:4]
            valid8 = m8 > 0.0
            ious_sc[pl.ds(g8, 8), :] = jnp.where(valid8, li8, 0.0)
            x8 = jnp.maximum(li8, 1e-8)
            ic8 = jnp.where(x8 == 1e-8, kconst, -jnp.log(x8))
            i38 = 3.0 * jnp.where(valid8, ic8, 0.0)
            tot8 = ((c3b + reg3(g8)) + i38) + (100000.0 *
                                               jnp.where(valid8, 0.0, 1.0))
            cost_ref[0, pl.ds(g8, 8), :] = tot8
            return 0

        jax.lax.fori_loop(0, T // 8, g_body, 0, unroll=False)

    lane_f = jax.lax.broadcasted_iota(jnp.int32, (T, P), 1).astype(jnp.float32)

    gm = jnp.where(allok, jnp.float32(0.0), jnp.max(ious_sc[:, :]))

    def a_body(s_, acc):
        iv = ious_sc[:, :]
        rm = jnp.max(iv, axis=1, keepdims=True)
        cand = jnp.where(iv == rm, lane_f, 1.0e9)
        am = jnp.min(cand, axis=1, keepdims=True)
        sel = lane_f == am
        ious_sc[:, :] = jnp.where(sel, -jnp.inf, iv)
        return acc + rm

    asteps = jnp.where(gm < 0.2, 0, 10)
    acc = jax.lax.fori_loop(0, asteps, a_body,
                            jnp.zeros((T, 1), jnp.float32))
    ks = jnp.clip(acc.astype(jnp.int32), 1, P)
    gv = (jax.lax.broadcasted_iota(jnp.int32, (T, 1), 0).astype(jnp.float32)
          < num_gt)
    kk_ref[0] = jnp.where(gv, ks, 0)


def _gather16(x, idx):
    dnums = lax.GatherDimensionNumbers(
        offset_dims=(), collapsed_slice_dims=(0,), start_index_map=(0,))
    return lax.gather(x, idx[:, None], dnums, (1,),
                      mode=lax.GatherScatterMode.PROMISE_IN_BOUNDS)


def _allmin16(x, iota16):
    for st in (1, 2, 4, 8):
        x = jnp.minimum(x, _gather16(x, iota16 ^ st))
    return x


def _sc_select(cost_ref, kk_ref, matched_ref, assigned_ref,
               buf0, buf1, kkb, mcost, match, asgb, sem0, sem1):
    wid = lax.axis_index("s") * 2 + lax.axis_index("c")
    iota16 = lax.iota(jnp.int32, 16)

    @pl.when(wid < B)
    def _worker():
        b = wid
        pltpu.sync_copy(kk_ref.at[pl.ds(b * T, T)], kkb.at[pl.ds(0, T)])

        def init_body(i, _):
            mcost[pl.ds(i * 16, 16)] = jnp.full((16,), BIG, jnp.float32)
            match[pl.ds(i * 16, 16)] = jnp.full((16,), -1, jnp.int32)
            return 0

        jax.lax.fori_loop(0, NCHUNK + 2, init_body, 0, unroll=8)

        bufs = (buf0, buf1)
        sems = (sem0, sem1)
        pltpu.make_async_copy(
            cost_ref.at[pl.ds(b * T, 8)], buf0, sem0).start()
        for w in range(4):
            buf = bufs[w % 2]
            pltpu.make_async_copy(
                cost_ref.at[pl.ds(b * T + w * 8, 8)],
                buf, sems[w % 2]).wait()
            if w < 3:
                pltpu.make_async_copy(
                    cost_ref.at[pl.ds(b * T + (w + 1) * 8, 8)],
                    bufs[(w + 1) % 2], sems[(w + 1) % 2]).start()

            def col_body(j, _, w=w, buf=buf):
                g = w * 8 + j
                kg = kkb[pl.ds(g, 16)][0]

                def round_body(r, carry):
                    pv, pi = carry

                    def chunk(i, c):
                        bv, bi = c
                        cv = buf[j, pl.ds(i * 16, 16)]
                        iv = iota16 + i * 16
                        ok = (cv > pv) | ((cv == pv) & (iv > pi))
                        cva = jnp.where(ok, cv, jnp.inf)
                        mlt = cva < bv
                        return (jnp.where(mlt, cva, bv),
                                jnp.where(mlt, iv, bi))

                    bv, bi = jax.lax.fori_loop(
                        0, NCHUNK, chunk,
                        (jnp.full((16,), jnp.inf, jnp.float32),
                         jnp.full((16,), P, jnp.int32)),
                        unroll=10)
                    mval = _allmin16(bv, iota16)
                    midx = _allmin16(jnp.where(bv == mval, bi, P), iota16)
                    mi = midx[0]
                    base = (mi // 16) * 16
                    lane = midx - base
                    cur16 = mcost[pl.ds(base, 16)]
                    updm = (iota16 == lane) & (mval < cur16)
                    mcost[pl.ds(base, 16)] = jnp.where(updm, mval, cur16)
                    mt16 = match[pl.ds(base, 16)]
                    match[pl.ds(base, 16)] = jnp.where(
                        updm, jnp.full((16,), g, jnp.int32), mt16)
                    return (mval[0], mi)

                jax.lax.fori_loop(
                    0, kg, round_body,
                    (jnp.float32(-jnp.inf), jnp.int32(-1)))
                return 0

            jax.lax.fori_loop(0, 8, col_body, 0)

        def out_body(i, _):
            mv = match[pl.ds(i * 16, 16)]
            asgb[pl.ds(i * 16, 16)] = jnp.where(
                mv >= 0, jnp.full((16,), 1, jnp.int32),
                jnp.full((16,), 0, jnp.int32))
            return 0

        jax.lax.fori_loop(0, NCHUNK, out_body, 0, unroll=8)
        pltpu.sync_copy(match.at[pl.ds(0, P)],
                        matched_ref.at[pl.ds(b * P, P)])
        pltpu.sync_copy(asgb, assigned_ref.at[pl.ds(b * P, P)])


@jax.jit
def _run(preds, targets, masks, img_w_f, wm1_f, kconst):
    logits_t = jnp.transpose(preds[:, :, 0:2], (0, 2, 1))
    reg_t = jnp.transpose(preds[:, :, 2:5], (0, 2, 1))
    lines_bf = jnp.transpose(preds[:, :, 6:], (0, 2, 1)).astype(jnp.bfloat16)
    tlines = targets[:, :, 6:].reshape(B, T, LINED, 1)
    tsc = jnp.stack(
        [targets[:, :, 3], targets[:, :, 2], targets[:, :, 4],
         masks.astype(jnp.float32)], axis=-1)
    masksv = masks.astype(jnp.float32).reshape(B, 1, T)
    scal = jnp.stack([wm1_f, img_w_f, kconst,
                      jnp.float32(0.0)]).reshape(1, 4)

    cost, kk3 = pl.pallas_call(
        _cost_body,
        grid=(B,),
        in_specs=[
            pl.BlockSpec((1, 4), lambda b: (0, 0), memory_space=pltpu.SMEM),
            pl.BlockSpec((1, 2, P), lambda b: (b, 0, 0)),
            pl.BlockSpec((1, 3, P), lambda b: (b, 0, 0)),
            pl.BlockSpec((1, LINED, P), lambda b: (b, 0, 0)),
            pl.BlockSpec((1, T, LINED, 1), lambda b: (b, 0, 0, 0)),
            pl.BlockSpec((1, T, 4), lambda b: (b, 0, 0)),
            pl.BlockSpec((1, 1, T), lambda b: (b, 0, 0)),
        ],
        out_specs=[
            pl.BlockSpec((1, T, P), lambda b: (b, 0, 0)),
            pl.BlockSpec((1, T, 1), lambda b: (b, 0, 0)),
        ],
        out_shape=[
            jax.ShapeDtypeStruct((B, T, P), jnp.float32),
            jax.ShapeDtypeStruct((B, T, 1), jnp.int32),
        ],
        scratch_shapes=[
            pltpu.VMEM((T, P), jnp.float32),
        ],
    )(scal, logits_t, reg_t, lines_bf, tlines, tsc, masksv)

    kk = kk3.reshape(B, T)

    mesh = plsc.VectorSubcoreMesh(core_axis_name="c", subcore_axis_name="s")
    sel = pl.kernel(
        _sc_select,
        mesh=mesh,
        out_type=[
            jax.ShapeDtypeStruct((B * P,), jnp.int32),
            jax.ShapeDtypeStruct((B * P,), jnp.int32),
        ],
        scratch_types=[
            pltpu.VMEM((8, P), jnp.float32),
            pltpu.VMEM((8, P), jnp.float32),
            pltpu.VMEM((T + 16,), jnp.int32),
            pltpu.VMEM((P + 32,), jnp.float32),
            pltpu.VMEM((P + 32,), jnp.int32),
            pltpu.VMEM((P,), jnp.int32),
            pltpu.SemaphoreType.DMA,
            pltpu.SemaphoreType.DMA,
        ],
    )
    matched, assigned = sel(cost.reshape(B * T, P), kk.reshape(-1))
    return assigned.reshape(B, P) != 0, matched.reshape(B, P)


def kernel(preds, targets, masks, img_w, img_h):
    img_w_f = jnp.asarray(img_w).astype(jnp.float32)
    wm1_f = (jnp.asarray(img_w) - 1).astype(jnp.float32)
    kconst = -jnp.log(jnp.clip(jnp.float32(1e-8), 1e-08, None))
    return _run(preds, targets, masks, img_w_f, wm1_f, kconst)


# speedup vs baseline: 1.1379x; 1.1379x over previous
import functools

import jax
import jax.numpy as jnp
from jax import lax
from jax.experimental import pallas as pl
from jax.experimental.pallas import tpu as pltpu
from jax.experimental.pallas import tpu_sc as plsc

B, P, T = 16, 4000, 32
LINED = 72
BIG = 100000000.0
NCHUNK = P // 16


def _cost_body(scal_ref, logits_ref, reg_ref, lines_ref, tlines_ref, tscv_ref,
               masksv_ref, cost_ref, kk_ref, ious_sc):
    wm1 = scal_ref[0, 0]
    wf = scal_ref[0, 1]
    kconst = scal_ref[0, 2]

    lg = logits_ref[0]
    x0 = lg[0:1, :]
    x1 = lg[1:2, :]
    m = jnp.maximum(x0, x1)
    e0 = jnp.exp(x0 - m)
    e1 = jnp.exp(x1 - m)
    p1 = e1 / (e0 + e1)
    c3 = 3.0 * (-jnp.log(jnp.maximum(p1, 1e-8)))

    p2 = reg_ref[0, 0:1, :]
    p3 = reg_ref[0, 1:2, :]
    p4 = reg_ref[0, 2:3, :]
    p2b = jnp.broadcast_to(p2, (8, P))
    p3b = jnp.broadcast_to(p3, (8, P))
    p4b = jnp.broadcast_to(p4, (8, P))
    c3b = jnp.broadcast_to(c3, (8, P))
    lb = lines_ref[0]

    num_gt = jnp.sum(masksv_ref[0])

    def g_body(gi, _):
        g8 = gi * 8
        s_rows = []
        nv_rows = []
        for j in range(8):
            gl = tlines_ref[0, g8 + j]
            gls = gl * wm1
            inv = (gls < 0.0) | (gls >= wf)
            v = jnp.where(inv, 0.0, 1.0)
            ad = jnp.abs(lb - gl.astype(jnp.bfloat16))
            sj = jax.lax.dot_general(
                v.astype(jnp.bfloat16), ad,
                (((0,), (0,)), ((), ())),
                preferred_element_type=jnp.float32)
            s_rows.append(sj)
            nv_rows.append(jnp.sum(v).reshape(1, 1))
        s8 = wm1 * jnp.concatenate(s_rows, axis=0)
        nv8 = jnp.concatenate(nv_rows, axis=0)
        a8 = 30.0 * nv8
        li8 = (a8 - s8) / ((a8 + s8) + 1e-9)
        m8 = tscv_ref[0, pl.ds(g8, 8), 3:4]
        valid8 = m8 > 0.0
        ious_sc[pl.ds(g8, 8), :] = jnp.where(valid8, li8, 0.0)
        x8 = jnp.maximum(li8, 1e-8)
        ic8 = jnp.where(x8 == 1e-8, kconst, -jnp.log(x8))
        i38 = 3.0 * jnp.where(valid8, ic8, 0.0)
        t3c = tscv_ref[0, pl.ds(g8, 8), 0:1]
        t2c = tscv_ref[0, pl.ds(g8, 8), 1:2]
        t4c = tscv_ref[0, pl.ds(g8, 8), 2:3]
        cx = jnp.abs(p3b - t3c)
        cy = jnp.abs(p2b - t2c)
        ct = jnp.abs(p4b - t4c)
        r38 = 3.0 * ((cx + cy) + ct)
        tot8 = ((c3b + r38) + i38) + (100000.0 * jnp.where(valid8, 0.0, 1.0))
        cost_ref[0, pl.ds(g8, 8), :] = tot8
        return 0

    jax.lax.fori_loop(0, T // 8, g_body, 0, unroll=False)

    lane_f = jax.lax.broadcasted_iota(jnp.int32, (T, P), 1).astype(jnp.float32)

    gm = jnp.max(ious_sc[:, :])

    def a_body(s_, acc):
        iv = ious_sc[:, :]
        rm = jnp.max(iv, axis=1, keepdims=True)
        cand = jnp.where(iv == rm, lane_f, 1.0e9)
        am = jnp.min(cand, axis=1, keepdims=True)
        sel = lane_f == am
        ious_sc[:, :] = jnp.where(sel, -jnp.inf, iv)
        return acc + rm

    asteps = jnp.where(gm < 0.2, 0, 10)
    acc = jax.lax.fori_loop(0, asteps, a_body,
                            jnp.zeros((T, 1), jnp.float32))
    ks = jnp.clip(acc.astype(jnp.int32), 1, P)
    gv = (jax.lax.broadcasted_iota(jnp.int32, (T, 1), 0).astype(jnp.float32)
          < num_gt)
    kk_ref[0] = jnp.where(gv, ks, 0)


def _gather16(x, idx):
    dnums = lax.GatherDimensionNumbers(
        offset_dims=(), collapsed_slice_dims=(0,), start_index_map=(0,))
    return lax.gather(x, idx[:, None], dnums, (1,),
                      mode=lax.GatherScatterMode.PROMISE_IN_BOUNDS)


def _allmin16(x, iota16):
    for st in (1, 2, 4, 8):
        x = jnp.minimum(x, _gather16(x, iota16 ^ st))
    return x


def _sc_select(cost_ref, kk_ref, matched_ref, assigned_ref,
               buf0, buf1, kkb, mcost, match, asgb, sem0, sem1):
    wid = lax.axis_index("s") * 2 + lax.axis_index("c")
    iota16 = lax.iota(jnp.int32, 16)

    @pl.when(wid < B)
    def _worker():
        b = wid
        pltpu.sync_copy(kk_ref.at[pl.ds(b * T, T)], kkb.at[pl.ds(0, T)])

        def init_body(i, _):
            mcost[pl.ds(i * 16, 16)] = jnp.full((16,), BIG, jnp.float32)
            match[pl.ds(i * 16, 16)] = jnp.full((16,), -1, jnp.int32)
            return 0

        jax.lax.fori_loop(0, NCHUNK + 2, init_body, 0, unroll=8)

        bufs = (buf0, buf1)
        sems = (sem0, sem1)
        pltpu.make_async_copy(
            cost_ref.at[pl.ds(b * T, 8)], buf0, sem0).start()
        for w in range(4):
            buf = bufs[w % 2]
            pltpu.make_async_copy(
                cost_ref.at[pl.ds(b * T + w * 8, 8)],
                buf, sems[w % 2]).wait()
            if w < 3:
                pltpu.make_async_copy(
                    cost_ref.at[pl.ds(b * T + (w + 1) * 8, 8)],
                    bufs[(w + 1) % 2], sems[(w + 1) % 2]).start()

            def col_body(j, _, w=w, buf=buf):
                g = w * 8 + j
                kg = kkb[pl.ds(g, 16)][0]

                def round_body(r, carry):
                    pv, pi = carry

                    def chunk(i, c):
                        bv, bi = c
                        cv = buf[j, pl.ds(i * 16, 16)]
                        iv = iota16 + i * 16
                        ok = (cv > pv) | ((cv == pv) & (iv > pi))
                        cva = jnp.where(ok, cv, jnp.inf)
                        mlt = cva < bv
                        return (jnp.where(mlt, cva, bv),
                                jnp.where(mlt, iv, bi))

                    bv, bi = jax.lax.fori_loop(
                        0, NCHUNK, chunk,
                        (jnp.full((16,), jnp.inf, jnp.float32),
                         jnp.full((16,), P, jnp.int32)),
                        unroll=10)
                    mval = _allmin16(bv, iota16)
                    midx = _allmin16(jnp.where(bv == mval, bi, P), iota16)
                    mi = midx[0]
                    base = (mi // 16) * 16
                    lane = midx - base
                    cur16 = mcost[pl.ds(base, 16)]
                    updm = (iota16 == lane) & (mval < cur16)
                    mcost[pl.ds(base, 16)] = jnp.where(updm, mval, cur16)
                    mt16 = match[pl.ds(base, 16)]
                    match[pl.ds(base, 16)] = jnp.where(
                        updm, jnp.full((16,), g, jnp.int32), mt16)
                    return (mval[0], mi)

                jax.lax.fori_loop(
                    0, kg, round_body,
                    (jnp.float32(-jnp.inf), jnp.int32(-1)))
                return 0

            jax.lax.fori_loop(0, 8, col_body, 0)

        def out_body(i, _):
            mv = match[pl.ds(i * 16, 16)]
            asgb[pl.ds(i * 16, 16)] = jnp.where(
                mv >= 0, jnp.full((16,), 1, jnp.int32),
                jnp.full((16,), 0, jnp.int32))
            return 0

        jax.lax.fori_loop(0, NCHUNK, out_body, 0, unroll=8)
        pltpu.sync_copy(match.at[pl.ds(0, P)],
                        matched_ref.at[pl.ds(b * P, P)])
        pltpu.sync_copy(asgb, assigned_ref.at[pl.ds(b * P, P)])


@jax.jit
def _run(preds, targets, masks, img_w_f, wm1_f, kconst):
    logits_t = jnp.transpose(preds[:, :, 0:2], (0, 2, 1))
    reg_t = jnp.transpose(preds[:, :, 2:5], (0, 2, 1))
    lines_bf = jnp.transpose(preds[:, :, 6:], (0, 2, 1)).astype(jnp.bfloat16)
    tlines = targets[:, :, 6:].reshape(B, T, LINED, 1)
    tsc = jnp.stack(
        [targets[:, :, 3], targets[:, :, 2], targets[:, :, 4],
         masks.astype(jnp.float32)], axis=-1)
    masksv = masks.astype(jnp.float32).reshape(B, 1, T)
    scal = jnp.stack([wm1_f, img_w_f, kconst,
                      jnp.float32(0.0)]).reshape(1, 4)

    cost, kk3 = pl.pallas_call(
        _cost_body,
        grid=(B,),
        in_specs=[
            pl.BlockSpec((1, 4), lambda b: (0, 0), memory_space=pltpu.SMEM),
            pl.BlockSpec((1, 2, P), lambda b: (b, 0, 0)),
            pl.BlockSpec((1, 3, P), lambda b: (b, 0, 0)),
            pl.BlockSpec((1, LINED, P), lambda b: (b, 0, 0)),
            pl.BlockSpec((1, T, LINED, 1), lambda b: (b, 0, 0, 0)),
            pl.BlockSpec((1, T, 4), lambda b: (b, 0, 0)),
            pl.BlockSpec((1, 1, T), lambda b: (b, 0, 0)),
        ],
        out_specs=[
            pl.BlockSpec((1, T, P), lambda b: (b, 0, 0)),
            pl.BlockSpec((1, T, 1), lambda b: (b, 0, 0)),
        ],
        out_shape=[
            jax.ShapeDtypeStruct((B, T, P), jnp.float32),
            jax.ShapeDtypeStruct((B, T, 1), jnp.int32),
        ],
        scratch_shapes=[
            pltpu.VMEM((T, P), jnp.float32),
        ],
    )(scal, logits_t, reg_t, lines_bf, tlines, tsc, masksv)

    kk = kk3.reshape(B, T)

    mesh = plsc.VectorSubcoreMesh(core_axis_name="c", subcore_axis_name="s")
    sel = pl.kernel(
        _sc_select,
        mesh=mesh,
        out_type=[
            jax.ShapeDtypeStruct((B * P,), jnp.int32),
            jax.ShapeDtypeStruct((B * P,), jnp.int32),
        ],
        scratch_types=[
            pltpu.VMEM((8, P), jnp.float32),
            pltpu.VMEM((8, P), jnp.float32),
            pltpu.VMEM((T + 16,), jnp.int32),
            pltpu.VMEM((P + 32,), jnp.float32),
            pltpu.VMEM((P + 32,), jnp.int32),
            pltpu.VMEM((P,), jnp.int32),
            pltpu.SemaphoreType.DMA,
            pltpu.SemaphoreType.DMA,
        ],
    )
    matched, assigned = sel(cost.reshape(B * T, P), kk.reshape(-1))
    return assigned.reshape(B, P) != 0, matched.reshape(B, P)


def kernel(preds, targets, masks, img_w, img_h):
    img_w_f = jnp.asarray(img_w).astype(jnp.float32)
    wm1_f = (jnp.asarray(img_w) - 1).astype(jnp.float32)
    kconst = -jnp.log(jnp.clip(jnp.float32(1e-8), 1e-08, None))
    return _run(preds, targets, masks, img_w_f, wm1_f, kconst)


# speedup vs baseline: 1.2615x; 1.1086x over previous
import functools

import jax
import jax.numpy as jnp
from jax import lax
from jax.experimental import pallas as pl
from jax.experimental.pallas import tpu as pltpu
from jax.experimental.pallas import tpu_sc as plsc

B, P, T = 16, 4000, 32
LINED = 72
BIG = 100000000.0
NCHUNK = P // 16


def _cost_body(scal_ref, logits_ref, reg_ref, lines_ref, tlines_ref, tscv_ref,
               masksv_ref, cost_ref, kk_ref, ious_sc):
    wm1 = scal_ref[0, 0]
    wf = scal_ref[0, 1]
    kconst = scal_ref[0, 2]

    lg = logits_ref[0]
    x0 = lg[0:1, :]
    x1 = lg[1:2, :]
    m = jnp.maximum(x0, x1)
    e0 = jnp.exp(x0 - m)
    e1 = jnp.exp(x1 - m)
    p1 = e1 / (e0 + e1)
    c3 = 3.0 * (-jnp.log(jnp.maximum(p1, 1e-8)))

    p2 = reg_ref[0, 0:1, :]
    p3 = reg_ref[0, 1:2, :]
    p4 = reg_ref[0, 2:3, :]
    p2b = jnp.broadcast_to(p2, (8, P))
    p3b = jnp.broadcast_to(p3, (8, P))
    p4b = jnp.broadcast_to(p4, (8, P))
    c3b = jnp.broadcast_to(c3, (8, P))
    lb = lines_ref[0]

    num_gt = jnp.sum(masksv_ref[0])

    def g_body(gi, _):
        g8 = gi * 8
        s_rows = []
        nv_rows = []
        for j in range(8):
            gl = tlines_ref[0, g8 + j]
            gls = gl * wm1
            inv = (gls < 0.0) | (gls >= wf)
            v = jnp.where(inv, 0.0, 1.0)
            ad = jnp.abs(lb - gl.astype(jnp.bfloat16))
            sj = jax.lax.dot_general(
                v.astype(jnp.bfloat16), ad,
                (((0,), (0,)), ((), ())),
                preferred_element_type=jnp.float32)
            s_rows.append(sj)
            nv_rows.append(jnp.sum(v).reshape(1, 1))
        s8 = wm1 * jnp.concatenate(s_rows, axis=0)
        nv8 = jnp.concatenate(nv_rows, axis=0)
        a8 = 30.0 * nv8
        li8 = (a8 - s8) / ((a8 + s8) + 1e-9)
        m8 = tscv_ref[0, pl.ds(g8, 8), 3:4]
        valid8 = m8 > 0.0
        ious_sc[pl.ds(g8, 8), :] = jnp.where(valid8, li8, 0.0)
        x8 = jnp.maximum(li8, 1e-8)
        ic8 = jnp.where(x8 == 1e-8, kconst, -jnp.log(x8))
        i38 = 3.0 * jnp.where(valid8, ic8, 0.0)
        t3c = tscv_ref[0, pl.ds(g8, 8), 0:1]
        t2c = tscv_ref[0, pl.ds(g8, 8), 1:2]
        t4c = tscv_ref[0, pl.ds(g8, 8), 2:3]
        cx = jnp.abs(p3b - t3c)
        cy = jnp.abs(p2b - t2c)
        ct = jnp.abs(p4b - t4c)
        r38 = 3.0 * ((cx + cy) + ct)
        tot8 = ((c3b + r38) + i38) + (100000.0 * jnp.where(valid8, 0.0, 1.0))
        cost_ref[0, pl.ds(g8, 8), :] = tot8
        return 0

    jax.lax.fori_loop(0, T // 8, g_body, 0, unroll=False)

    lane_f = jax.lax.broadcasted_iota(jnp.int32, (T, P), 1).astype(jnp.float32)

    gm = jnp.max(ious_sc[:, :])

    def a_body(s_, acc):
        iv = ious_sc[:, :]
        rm = jnp.max(iv, axis=1, keepdims=True)
        cand = jnp.where(iv == rm, lane_f, 1.0e9)
        am = jnp.min(cand, axis=1, keepdims=True)
        sel = lane_f == am
        ious_sc[:, :] = jnp.where(sel, -jnp.inf, iv)
        return acc + rm

    asteps = jnp.where(gm < 0.2, 0, 10)
    acc = jax.lax.fori_loop(0, asteps, a_body,
                            jnp.zeros((T, 1), jnp.float32))
    ks = jnp.clip(acc.astype(jnp.int32), 1, P)
    gv = (jax.lax.broadcasted_iota(jnp.int32, (T, 1), 0).astype(jnp.float32)
          < num_gt)
    kk_ref[0] = jnp.where(gv, ks, 0)


def _gather16(x, idx):
    dnums = lax.GatherDimensionNumbers(
        offset_dims=(), collapsed_slice_dims=(0,), start_index_map=(0,))
    return lax.gather(x, idx[:, None], dnums, (1,),
                      mode=lax.GatherScatterMode.PROMISE_IN_BOUNDS)


def _allmin16(x, iota16):
    for st in (1, 2, 4, 8):
        x = jnp.minimum(x, _gather16(x, iota16 ^ st))
    return x


def _sc_select(cost_ref, kk_ref, matched_ref, assigned_ref,
               buf0, buf1, kkb, mcost, match, asgb,
               candc, candi, pcandc, pcandi, sharedc, sharedi, sem0, sem1):
    cidx = lax.axis_index("c")
    sidx = lax.axis_index("s")
    b = cidx * 8 + sidx // 2
    half = sidx % 2
    g0off = half * 16
    iota16 = lax.iota(jnp.int32, 16)

    pltpu.sync_copy(kk_ref.at[pl.ds(b * T, T)], kkb.at[pl.ds(0, T)])

    bufs = (buf0, buf1)
    sems = (sem0, sem1)
    pltpu.make_async_copy(
        cost_ref.at[pl.ds(b * T + g0off, 8)], buf0, sem0).start()
    for w in range(2):
        buf = bufs[w % 2]
        pltpu.make_async_copy(
            cost_ref.at[pl.ds(b * T + g0off + w * 8, 8)],
            buf, sems[w % 2]).wait()
        if w < 1:
            pltpu.make_async_copy(
                cost_ref.at[pl.ds(b * T + g0off + (w + 1) * 8, 8)],
                bufs[(w + 1) % 2], sems[(w + 1) % 2]).start()

        def col_body(j, _, w=w, buf=buf):
            g = g0off + w * 8 + j
            kg = kkb[pl.ds(g, 16)][0]
            cbase = (w * 8 + j) * 16

            def round_body(r, carry):
                pv, pi = carry

                def chunk(i, c):
                    bv, bi = c
                    cv = buf[j, pl.ds(i * 16, 16)]
                    iv = iota16 + i * 16
                    ok = (cv > pv) | ((cv == pv) & (iv > pi))
                    cva = jnp.where(ok, cv, jnp.inf)
                    mlt = cva < bv
                    return (jnp.where(mlt, cva, bv),
                            jnp.where(mlt, iv, bi))

                bv, bi = jax.lax.fori_loop(
                    0, NCHUNK, chunk,
                    (jnp.full((16,), jnp.inf, jnp.float32),
                     jnp.full((16,), P, jnp.int32)),
                    unroll=10)
                mval = _allmin16(bv, iota16)
                midx = _allmin16(jnp.where(bv == mval, bi, P), iota16)
                selr = iota16 == r
                cc16 = candc[pl.ds(cbase, 16)]
                candc[pl.ds(cbase, 16)] = jnp.where(selr, mval, cc16)
                ci16 = candi[pl.ds(cbase, 16)]
                candi[pl.ds(cbase, 16)] = jnp.where(selr, midx, ci16)
                return (mval[0], midx[0])

            jax.lax.fori_loop(
                0, kg, round_body,
                (jnp.float32(-jnp.inf), jnp.int32(-1)))
            return 0

        jax.lax.fori_loop(0, 8, col_body, 0)

    @pl.when(half == 1)
    def _publish():
        pltpu.sync_copy(candc.at[pl.ds(0, 256)],
                        sharedc.at[pl.ds(sidx * 256, 256)])
        pltpu.sync_copy(candi.at[pl.ds(0, 256)],
                        sharedi.at[pl.ds(sidx * 256, 256)])

    plsc.subcore_barrier()

    @pl.when(half == 0)
    def _merge():
        pltpu.sync_copy(sharedc.at[pl.ds((sidx + 1) * 256, 256)],
                        pcandc.at[pl.ds(0, 256)])
        pltpu.sync_copy(sharedi.at[pl.ds((sidx + 1) * 256, 256)],
                        pcandi.at[pl.ds(0, 256)])

        def init_body(i, _):
            mcost[pl.ds(i * 16, 16)] = jnp.full((16,), BIG, jnp.float32)
            match[pl.ds(i * 16, 16)] = jnp.full((16,), -1, jnp.int32)
            return 0

        jax.lax.fori_loop(0, NCHUNK + 2, init_body, 0, unroll=8)

        def merge_from(cc_arr, ci_arr, gbase):
            def mg_body(gl, _):
                g = gbase + gl
                kg = kkb[pl.ds(g, 16)][0]

                def r_body(r, _):
                    mval = cc_arr[pl.ds(gl * 16 + r, 16)][0]
                    mi = ci_arr[pl.ds(gl * 16 + r, 16)][0]
                    base = (mi // 16) * 16
                    lanepos = mi - base
                    cur16 = mcost[pl.ds(base, 16)]
                    updm = (iota16 == lanepos) & (mval < cur16)
                    mcost[pl.ds(base, 16)] = jnp.where(updm, mval, cur16)
                    mt16 = match[pl.ds(base, 16)]
                    match[pl.ds(base, 16)] = jnp.where(
                        updm, jnp.full((16,), g, jnp.int32), mt16)
                    return 0

                jax.lax.fori_loop(0, kg, r_body, 0)
                return 0

            jax.lax.fori_loop(0, 16, mg_body, 0)

        merge_from(candc, candi, 0)
        merge_from(pcandc, pcandi, 16)

        def out_body(i, _):
            mv = match[pl.ds(i * 16, 16)]
            asgb[pl.ds(i * 16, 16)] = jnp.where(
                mv >= 0, jnp.full((16,), 1, jnp.int32),
                jnp.full((16,), 0, jnp.int32))
            return 0

        jax.lax.fori_loop(0, NCHUNK, out_body, 0, unroll=8)
        pltpu.sync_copy(match.at[pl.ds(0, P)],
                        matched_ref.at[pl.ds(b * P, P)])
        pltpu.sync_copy(asgb, assigned_ref.at[pl.ds(b * P, P)])


@jax.jit
def _run(preds, targets, masks, img_w_f, wm1_f, kconst):
    logits_t = jnp.transpose(preds[:, :, 0:2], (0, 2, 1))
    reg_t = jnp.transpose(preds[:, :, 2:5], (0, 2, 1))
    lines_bf = jnp.transpose(preds[:, :, 6:], (0, 2, 1)).astype(jnp.bfloat16)
    tlines = targets[:, :, 6:].reshape(B, T, LINED, 1)
    tsc = jnp.stack(
        [targets[:, :, 3], targets[:, :, 2], targets[:, :, 4],
         masks.astype(jnp.float32)], axis=-1)
    masksv = masks.astype(jnp.float32).reshape(B, 1, T)
    scal = jnp.stack([wm1_f, img_w_f, kconst,
                      jnp.float32(0.0)]).reshape(1, 4)

    cost, kk3 = pl.pallas_call(
        _cost_body,
        grid=(B,),
        in_specs=[
            pl.BlockSpec((1, 4), lambda b: (0, 0), memory_space=pltpu.SMEM),
            pl.BlockSpec((1, 2, P), lambda b: (b, 0, 0)),
            pl.BlockSpec((1, 3, P), lambda b: (b, 0, 0)),
            pl.BlockSpec((1, LINED, P), lambda b: (b, 0, 0)),
            pl.BlockSpec((1, T, LINED, 1), lambda b: (b, 0, 0, 0)),
            pl.BlockSpec((1, T, 4), lambda b: (b, 0, 0)),
            pl.BlockSpec((1, 1, T), lambda b: (b, 0, 0)),
        ],
        out_specs=[
            pl.BlockSpec((1, T, P), lambda b: (b, 0, 0)),
            pl.BlockSpec((1, T, 1), lambda b: (b, 0, 0)),
        ],
        out_shape=[
            jax.ShapeDtypeStruct((B, T, P), jnp.float32),
            jax.ShapeDtypeStruct((B, T, 1), jnp.int32),
        ],
        scratch_shapes=[
            pltpu.VMEM((T, P), jnp.float32),
        ],
    )(scal, logits_t, reg_t, lines_bf, tlines, tsc, masksv)

    kk = kk3.reshape(B, T)

    mesh = plsc.VectorSubcoreMesh(core_axis_name="c", subcore_axis_name="s")
    sel = pl.kernel(
        _sc_select,
        mesh=mesh,
        out_type=[
            jax.ShapeDtypeStruct((B * P,), jnp.int32),
            jax.ShapeDtypeStruct((B * P,), jnp.int32),
        ],
        scratch_types=[
            pltpu.VMEM((8, P), jnp.float32),
            pltpu.VMEM((8, P), jnp.float32),
            pltpu.VMEM((T + 16,), jnp.int32),
            pltpu.VMEM((P + 32,), jnp.float32),
            pltpu.VMEM((P + 32,), jnp.int32),
            pltpu.VMEM((P,), jnp.int32),
            pltpu.VMEM((288,), jnp.float32),
            pltpu.VMEM((288,), jnp.int32),
            pltpu.VMEM((288,), jnp.float32),
            pltpu.VMEM((288,), jnp.int32),
            pltpu.VMEM_SHARED((4096,), jnp.float32),
            pltpu.VMEM_SHARED((4096,), jnp.int32),
            pltpu.SemaphoreType.DMA,
            pltpu.SemaphoreType.DMA,
        ],
    )
    matched, assigned = sel(cost.reshape(B * T, P), kk.reshape(-1))
    return assigned.reshape(B, P) != 0, matched.reshape(B, P)


def kernel(preds, targets, masks, img_w, img_h):
    img_w_f = jnp.asarray(img_w).astype(jnp.float32)
    wm1_f = (jnp.asarray(img_w) - 1).astype(jnp.float32)
    kconst = -jnp.log(jnp.clip(jnp.float32(1e-8), 1e-08, None))
    return _run(preds, targets, masks, img_w_f, wm1_f, kconst)
